# baseline (speedup 1.0000x reference)
import functools

import jax
import jax.numpy as jnp
from jax import lax
from jax.experimental import pallas as pl
from jax.experimental.pallas import tpu as pltpu
from jax.experimental.pallas import tpu_sc as plsc

N = 10000
E = 320000
H = 128
G = 64

N_PAD = 10240
E_PAD = 327680
CHUNK = 128
N_WORKERS = 32
CHUNKS_PER_W = E_PAD // (CHUNK * N_WORKERS)
AGG_ROWS = 10112
AGG_PER_TILE = AGG_ROWS // 16
POOL_ROWS = N_PAD // N_WORKERS
ACC_G = 72
TRASH = N

FLT_MIN = jnp.finfo(jnp.float32).min


def _enc_body(x_ref, w_ref, b_ref, o_ref):
    o_ref[...] = jnp.maximum(
        jnp.dot(x_ref[...], w_ref[...], preferred_element_type=jnp.float32)
        + b_ref[...], 0.0)


def _encode(x, w, b, block_rows):
    rows, k = x.shape
    h_out = w.shape[1]
    grid = rows // block_rows
    return pl.pallas_call(
        _enc_body,
        grid=(grid,),
        in_specs=[
            pl.BlockSpec((block_rows, k), lambda i: (i, 0)),
            pl.BlockSpec((k, h_out), lambda i: (0, 0)),
            pl.BlockSpec((1, h_out), lambda i: (0, 0)),
        ],
        out_specs=pl.BlockSpec((block_rows, h_out), lambda i: (i, 0)),
        out_shape=jax.ShapeDtypeStruct((rows, h_out), jnp.float32),
    )(x, w, b.reshape(1, h_out))


def _mlp_body(h_ref, agg_ref, w1_ref, b1_ref, w2_ref, b2_ref, o_ref):
    z = h_ref[...] + agg_ref[0] + agg_ref[1]
    t = jnp.maximum(
        jnp.dot(z, w1_ref[...], preferred_element_type=jnp.float32)
        + b1_ref[...], 0.0)
    o_ref[...] = jnp.maximum(
        jnp.dot(t, w2_ref[...], preferred_element_type=jnp.float32)
        + b2_ref[...], 0.0)


def _conv_mlp(h, agg, w1, b1, w2, b2):
    block = 2048
    grid = N_PAD // block
    return pl.pallas_call(
        _mlp_body,
        grid=(grid,),
        in_specs=[
            pl.BlockSpec((block, H), lambda i: (i, 0)),
            pl.BlockSpec((2, block, H), lambda i: (0, i, 0)),
            pl.BlockSpec((H, H), lambda i: (0, 0)),
            pl.BlockSpec((1, H), lambda i: (0, 0)),
            pl.BlockSpec((H, H), lambda i: (0, 0)),
            pl.BlockSpec((1, H), lambda i: (0, 0)),
        ],
        out_specs=pl.BlockSpec((block, H), lambda i: (i, 0)),
        out_shape=jax.ShapeDtypeStruct((N_PAD, H), jnp.float32),
    )(h, agg, w1, b1.reshape(1, H), w2, b2.reshape(1, H))


def _head_body(ps_ref, pm_ref, pc_ref, w1_ref, b1_ref, w2_ref, b2_ref, o_ref):
    s = jnp.sum(ps_ref[...], axis=0)[:G]
    m = jnp.max(pm_ref[...], axis=0)[:G]
    cnt = jnp.sum(pc_ref[...], axis=0)[:G, :1]
    mean = s / jnp.maximum(cnt, 1.0)
    pooled = jnp.concatenate([mean, m], axis=1)
    t = jnp.maximum(
        jnp.dot(pooled, w1_ref[...], preferred_element_type=jnp.float32)
        + b1_ref[...], 0.0)
    o_ref[...] = (
        jnp.dot(t, w2_ref[...], preferred_element_type=jnp.float32)
        + b2_ref[...])


def _head(psum, pmax, pcnt, h1_w, h1_b, h2_w, h2_b):
    return pl.pallas_call(
        _head_body,
        out_shape=jax.ShapeDtypeStruct((G, 1), jnp.float32),
    )(psum, pmax, pcnt, h1_w, h1_b.reshape(1, H), h2_w, h2_b.reshape(1, 1))


@functools.lru_cache(maxsize=None)
def _sc_mesh():
    return plsc.VectorSubcoreMesh(core_axis_name="c", subcore_axis_name="s")


def _msg_body(h_hbm, e_hbm, idx_hbm, out_hbm,
              idxb0, idxb1, rows0, rows1, ebuf, agg, semg0, semg1):
    cid = lax.axis_index("c")
    sid = lax.axis_index("s")
    wid = sid * 2 + cid

    def _zrow(i, _):
        for j in range(8):
            rows0[i, pl.ds(j * 16, 16)] = jnp.zeros((16,), jnp.float32)
        return 0
    lax.fori_loop(0, CHUNK, _zrow, 0)

    zbase = sid * AGG_PER_TILE
    for k in range(4):
        pltpu.sync_copy(rows0, agg.at[pl.ds(zbase + k * CHUNK, CHUNK)])
    pltpu.sync_copy(rows0.at[pl.ds(0, AGG_PER_TILE - 4 * CHUNK)],
                    agg.at[pl.ds(zbase + 4 * CHUNK,
                                 AGG_PER_TILE - 4 * CHUNK)])
    plsc.subcore_barrier()

    cbase = wid * CHUNKS_PER_W
    pltpu.sync_copy(idx_hbm.at[cbase], idxb0)
    pltpu.async_copy(h_hbm.at[idxb0.at[0]], rows0, semg0)

    def _chunk(c, idxb_k, idxb_k1, rows_k, rows_k1, semg_k, semg_k1):
        @pl.when(c + 1 < CHUNKS_PER_W)
        def _prefetch():
            pltpu.sync_copy(idx_hbm.at[cbase + c + 1], idxb_k1)
            pltpu.async_copy(h_hbm.at[idxb_k1.at[0]], rows_k1, semg_k1)

        pltpu.make_async_copy(h_hbm.at[idxb_k.at[0]], rows_k, semg_k).wait()
        pltpu.sync_copy(e_hbm.at[pl.ds((cbase + c) * CHUNK, CHUNK)], ebuf)

        def _row(i, _):
            for j in range(8):
                sl = pl.ds(j * 16, 16)
                rows_k[i, sl] = jnp.maximum(rows_k[i, sl] + ebuf[i, sl], 0.0)
            return 0
        lax.fori_loop(0, CHUNK, _row, 0)

        pltpu.sync_copy(rows_k, agg.at[idxb_k.at[1]], add=True)

    def _pair(i, _):
        _chunk(2 * i, idxb0, idxb1, rows0, rows1, semg0, semg1)
        _chunk(2 * i + 1, idxb1, idxb0, rows1, rows0, semg1, semg0)
        return 0
    lax.fori_loop(0, CHUNKS_PER_W // 2, _pair, 0)
    plsc.subcore_barrier()

    pltpu.sync_copy(agg.at[pl.ds(sid * AGG_PER_TILE, AGG_PER_TILE)],
                    out_hbm.at[cid, pl.ds(sid * AGG_PER_TILE, AGG_PER_TILE)])


@functools.lru_cache(maxsize=None)
def _sc_message_kernel():
    return pl.kernel(
        _msg_body,
        out_type=jax.ShapeDtypeStruct((2, N_PAD, H), jnp.float32),
        mesh=_sc_mesh(),
        scratch_types=[
            pltpu.VMEM((2, CHUNK), jnp.int32),
            pltpu.VMEM((2, CHUNK), jnp.int32),
            pltpu.VMEM((CHUNK, H), jnp.float32),
            pltpu.VMEM((CHUNK, H), jnp.float32),
            pltpu.VMEM((CHUNK, H), jnp.float32),
            pltpu.VMEM_SHARED((AGG_ROWS, H), jnp.float32),
            pltpu.SemaphoreType.DMA,
            pltpu.SemaphoreType.DMA,
        ],
    )


def _sc_message(h, e, idx_both):
    return _sc_message_kernel()(h, e, idx_both)


def _pool_body(h_hbm, b_hbm, os_hbm, om_hbm, oc_hbm,
               hbuf, bbuf, asum, amax, acnt, sem):
    cid = lax.axis_index("c")
    sid = lax.axis_index("s")
    wid = sid * 2 + cid

    def _zacc(i, _):
        for j in range(8):
            sl = pl.ds(j * 16, 16)
            asum[i, sl] = jnp.zeros((16,), jnp.float32)
            amax[i, sl] = jnp.full((16,), FLT_MIN, jnp.float32)
        acnt[i, :] = jnp.zeros((16,), jnp.float32)
        return 0
    lax.fori_loop(0, ACC_G, _zacc, 0)

    base = wid * POOL_ROWS
    pltpu.sync_copy(b_hbm.at[pl.ds(base, POOL_ROWS)], bbuf.at[pl.ds(0, POOL_ROWS)])
    pltpu.async_copy(h_hbm.at[pl.ds(base, POOL_ROWS)], hbuf, sem).wait()

    def _row(i, _):
        g = bbuf[pl.ds(i, 16)][0]
        for j in range(8):
            sl = pl.ds(j * 16, 16)
            v = hbuf[i, sl]
            asum[g, sl] = asum[g, sl] + v
            amax[g, sl] = jnp.maximum(amax[g, sl], v)
        acnt[g, :] = acnt[g, :] + 1.0
        return 0
    lax.fori_loop(0, POOL_ROWS, _row, 0)

    pltpu.sync_copy(asum, os_hbm.at[wid])
    pltpu.sync_copy(amax, om_hbm.at[wid])
    pltpu.sync_copy(acnt, oc_hbm.at[wid])


@functools.lru_cache(maxsize=None)
def _sc_pool_kernel():
    return pl.kernel(
        _pool_body,
        out_type=(
            jax.ShapeDtypeStruct((N_WORKERS, ACC_G, H), jnp.float32),
            jax.ShapeDtypeStruct((N_WORKERS, ACC_G, H), jnp.float32),
            jax.ShapeDtypeStruct((N_WORKERS, ACC_G, 16), jnp.float32),
        ),
        mesh=_sc_mesh(),
        scratch_types=[
            pltpu.VMEM((POOL_ROWS, H), jnp.float32),
            pltpu.VMEM((POOL_ROWS + 16,), jnp.int32),
            pltpu.VMEM((ACC_G, H), jnp.float32),
            pltpu.VMEM((ACC_G, H), jnp.float32),
            pltpu.VMEM((ACC_G, 16), jnp.float32),
            pltpu.SemaphoreType.DMA,
        ],
    )


def _sc_pool(h, batch_pad):
    return _sc_pool_kernel()(h, batch_pad)


def kernel(x, edge_index, edge_attr, batch, w_node, b_node, w_edge, b_edge,
           c0_w1, c0_b1, c0_w2, c0_b2, c1_w1, c1_b1, c1_w2, c1_b2,
           c2_w1, c2_b1, c2_w2, c2_b2, h1_w, h1_b, h2_w, h2_b):
    x_pad = jnp.zeros((N_PAD, x.shape[1]), x.dtype).at[:N].set(x)
    batch_pad = jnp.full((N_PAD,), G, jnp.int32).at[:N].set(batch)
    src_pad = jnp.zeros((E_PAD,), jnp.int32).at[:E].set(edge_index[0])
    dst_pad = jnp.full((E_PAD,), TRASH, jnp.int32).at[:E].set(edge_index[1])
    idx_both = jnp.stack([src_pad.reshape(-1, CHUNK),
                          dst_pad.reshape(-1, CHUNK)], axis=1)
    ea_pad = jnp.zeros((E_PAD, edge_attr.shape[1]), edge_attr.dtype
                       ).at[:E].set(edge_attr)

    h = _encode(x_pad, w_node, b_node, 2048)
    e = _encode(ea_pad, w_edge, b_edge, 2048)

    for (w1, b1, w2, b2) in ((c0_w1, c0_b1, c0_w2, c0_b2),
                             (c1_w1, c1_b1, c1_w2, c1_b2),
                             (c2_w1, c2_b1, c2_w2, c2_b2)):
        agg = _sc_message(h, e, idx_both)
        h = _conv_mlp(h, agg, w1, b1, w2, b2)

    psum, pmax, pcnt = _sc_pool(h, batch_pad)
    out = _head(psum, pmax, pcnt, h1_w, h1_b, h2_w, h2_b)
    return out.reshape(-1)


# speedup vs baseline: 1.0604x; 1.0604x over previous
import functools

import jax
import jax.numpy as jnp
from jax import lax
from jax.experimental import pallas as pl
from jax.experimental.pallas import tpu as pltpu
from jax.experimental.pallas import tpu_sc as plsc

N = 10000
E = 320000
H = 128
G = 64

N_PAD = 10240
E_PAD = 323584
CHUNK = 128
N_WORKERS = 32
CHUNKS_PER_W = E_PAD // (CHUNK * N_WORKERS)
ROWS_PER_TILE = N_PAD // 16
POOL_ROWS = N_PAD // N_WORKERS
ACC_G = 72
TRASH = N

FLT_MIN = jnp.finfo(jnp.float32).min


def _enc_body(x_ref, w_ref, b_ref, o_ref):
    o_ref[...] = jnp.maximum(
        jnp.dot(x_ref[...], w_ref[...], preferred_element_type=jnp.float32)
        + b_ref[...], 0.0)


def _encode(x, w, b, block_rows):
    rows, k = x.shape
    h_out = w.shape[1]
    grid = rows // block_rows
    return pl.pallas_call(
        _enc_body,
        grid=(grid,),
        in_specs=[
            pl.BlockSpec((block_rows, k), lambda i: (i, 0)),
            pl.BlockSpec((k, h_out), lambda i: (0, 0)),
            pl.BlockSpec((1, h_out), lambda i: (0, 0)),
        ],
        out_specs=pl.BlockSpec((block_rows, h_out), lambda i: (i, 0)),
        out_shape=jax.ShapeDtypeStruct((rows, h_out), jnp.float32),
    )(x, w, b.reshape(1, h_out))


def _mlp_body(h_ref, agg_ref, w1_ref, b1_ref, w2_ref, b2_ref, o_ref):
    z = h_ref[...] + agg_ref[0] + agg_ref[1]
    t = jnp.maximum(
        jnp.dot(z, w1_ref[...], preferred_element_type=jnp.float32)
        + b1_ref[...], 0.0)
    o_ref[...] = jnp.maximum(
        jnp.dot(t, w2_ref[...], preferred_element_type=jnp.float32)
        + b2_ref[...], 0.0)


def _conv_mlp(h, agg, w1, b1, w2, b2):
    block = 2048
    grid = N_PAD // block
    return pl.pallas_call(
        _mlp_body,
        grid=(grid,),
        in_specs=[
            pl.BlockSpec((block, H), lambda i: (i, 0)),
            pl.BlockSpec((2, block, H), lambda i: (0, i, 0)),
            pl.BlockSpec((H, H), lambda i: (0, 0)),
            pl.BlockSpec((1, H), lambda i: (0, 0)),
            pl.BlockSpec((H, H), lambda i: (0, 0)),
            pl.BlockSpec((1, H), lambda i: (0, 0)),
        ],
        out_specs=pl.BlockSpec((block, H), lambda i: (i, 0)),
        out_shape=jax.ShapeDtypeStruct((N_PAD, H), jnp.float32),
    )(h, agg, w1, b1.reshape(1, H), w2, b2.reshape(1, H))


def _head_body(ps_ref, pm_ref, pc_ref, w1_ref, b1_ref, w2_ref, b2_ref, o_ref):
    s = jnp.sum(ps_ref[...], axis=0)[:G]
    m = jnp.max(pm_ref[...], axis=0)[:G]
    cnt = jnp.sum(pc_ref[...], axis=0)[:G, :1]
    mean = s / jnp.maximum(cnt, 1.0)
    pooled = jnp.concatenate([mean, m], axis=1)
    t = jnp.maximum(
        jnp.dot(pooled, w1_ref[...], preferred_element_type=jnp.float32)
        + b1_ref[...], 0.0)
    o_ref[...] = (
        jnp.dot(t, w2_ref[...], preferred_element_type=jnp.float32)
        + b2_ref[...])


def _head(psum, pmax, pcnt, h1_w, h1_b, h2_w, h2_b):
    return pl.pallas_call(
        _head_body,
        out_shape=jax.ShapeDtypeStruct((G, 1), jnp.float32),
    )(psum, pmax, pcnt, h1_w, h1_b.reshape(1, H), h2_w, h2_b.reshape(1, 1))


@functools.lru_cache(maxsize=None)
def _sc_mesh():
    return plsc.VectorSubcoreMesh(core_axis_name="c", subcore_axis_name="s")


def _msg_body(h_hbm, e_hbm, src_hbm, dst_hbm, out_hbm,
              idx_s, idx_d, rows, ebuf, agg, sem):
    cid = lax.axis_index("c")
    sid = lax.axis_index("s")
    wid = sid * 2 + cid

    def _zrow(i, _):
        for j in range(8):
            rows[i, pl.ds(j * 16, 16)] = jnp.zeros((16,), jnp.float32)
        return 0
    lax.fori_loop(0, CHUNK, _zrow, 0)

    def _zslab(k, _):
        pltpu.sync_copy(rows, agg.at[pl.ds(sid * ROWS_PER_TILE + k * CHUNK,
                                           CHUNK)])
        return 0
    lax.fori_loop(0, ROWS_PER_TILE // CHUNK, _zslab, 0)
    plsc.subcore_barrier()

    def _chunk(c, _):
        base = (wid * CHUNKS_PER_W + c) * CHUNK
        pltpu.sync_copy(src_hbm.at[pl.ds(base, CHUNK)], idx_s)
        pltpu.sync_copy(dst_hbm.at[pl.ds(base, CHUNK)], idx_d)
        pltpu.async_copy(h_hbm.at[idx_s], rows, sem).wait()
        pltpu.sync_copy(e_hbm.at[pl.ds(base, CHUNK)], ebuf)

        def _row(i, _):
            for j in range(8):
                sl = pl.ds(j * 16, 16)
                rows[i, sl] = jnp.maximum(rows[i, sl] + ebuf[i, sl], 0.0)
            return 0
        lax.fori_loop(0, CHUNK, _row, 0)

        pltpu.sync_copy(rows, agg.at[idx_d], add=True)
        return 0
    lax.fori_loop(0, CHUNKS_PER_W, _chunk, 0)
    plsc.subcore_barrier()

    pltpu.sync_copy(agg.at[pl.ds(sid * ROWS_PER_TILE, ROWS_PER_TILE)],
                    out_hbm.at[cid, pl.ds(sid * ROWS_PER_TILE, ROWS_PER_TILE)])


@functools.lru_cache(maxsize=None)
def _sc_message_kernel():
    return pl.kernel(
        _msg_body,
        out_type=jax.ShapeDtypeStruct((2, N_PAD, H), jnp.float32),
        mesh=_sc_mesh(),
        scratch_types=[
            pltpu.VMEM((CHUNK,), jnp.int32),
            pltpu.VMEM((CHUNK,), jnp.int32),
            pltpu.VMEM((CHUNK, H), jnp.float32),
            pltpu.VMEM((CHUNK, H), jnp.float32),
            pltpu.VMEM_SHARED((N_PAD, H), jnp.float32),
            pltpu.SemaphoreType.DMA,
        ],
    )


def _sc_message(h, e, src, dst):
    return _sc_message_kernel()(h, e, src, dst)


def _pool_body(h_hbm, b_hbm, os_hbm, om_hbm, oc_hbm,
               hbuf, bbuf, asum, amax, acnt, sem):
    cid = lax.axis_index("c")
    sid = lax.axis_index("s")
    wid = sid * 2 + cid

    def _zacc(i, _):
        for j in range(8):
            sl = pl.ds(j * 16, 16)
            asum[i, sl] = jnp.zeros((16,), jnp.float32)
            amax[i, sl] = jnp.full((16,), FLT_MIN, jnp.float32)
        acnt[i, :] = jnp.zeros((16,), jnp.float32)
        return 0
    lax.fori_loop(0, ACC_G, _zacc, 0)

    base = wid * POOL_ROWS
    pltpu.sync_copy(b_hbm.at[pl.ds(base, POOL_ROWS)], bbuf.at[pl.ds(0, POOL_ROWS)])
    pltpu.async_copy(h_hbm.at[pl.ds(base, POOL_ROWS)], hbuf, sem).wait()

    def _row(i, _):
        g = bbuf[pl.ds(i, 16)][0]
        for j in range(8):
            sl = pl.ds(j * 16, 16)
            v = hbuf[i, sl]
            asum[g, sl] = asum[g, sl] + v
            amax[g, sl] = jnp.maximum(amax[g, sl], v)
        acnt[g, :] = acnt[g, :] + 1.0
        return 0
    lax.fori_loop(0, POOL_ROWS, _row, 0)

    pltpu.sync_copy(asum, os_hbm.at[wid])
    pltpu.sync_copy(amax, om_hbm.at[wid])
    pltpu.sync_copy(acnt, oc_hbm.at[wid])


@functools.lru_cache(maxsize=None)
def _sc_pool_kernel():
    return pl.kernel(
        _pool_body,
        out_type=(
            jax.ShapeDtypeStruct((N_WORKERS, ACC_G, H), jnp.float32),
            jax.ShapeDtypeStruct((N_WORKERS, ACC_G, H), jnp.float32),
            jax.ShapeDtypeStruct((N_WORKERS, ACC_G, 16), jnp.float32),
        ),
        mesh=_sc_mesh(),
        scratch_types=[
            pltpu.VMEM((POOL_ROWS, H), jnp.float32),
            pltpu.VMEM((POOL_ROWS + 16,), jnp.int32),
            pltpu.VMEM((ACC_G, H), jnp.float32),
            pltpu.VMEM((ACC_G, H), jnp.float32),
            pltpu.VMEM((ACC_G, 16), jnp.float32),
            pltpu.SemaphoreType.DMA,
        ],
    )


def _sc_pool(h, batch_pad):
    return _sc_pool_kernel()(h, batch_pad)


def kernel(x, edge_index, edge_attr, batch, w_node, b_node, w_edge, b_edge,
           c0_w1, c0_b1, c0_w2, c0_b2, c1_w1, c1_b1, c1_w2, c1_b2,
           c2_w1, c2_b1, c2_w2, c2_b2, h1_w, h1_b, h2_w, h2_b):
    x_pad = jnp.zeros((N_PAD, x.shape[1]), x.dtype).at[:N].set(x)
    batch_pad = jnp.full((N_PAD,), G, jnp.int32).at[:N].set(batch)
    src_pad = jnp.zeros((E_PAD,), jnp.int32).at[:E].set(edge_index[0])
    dst_pad = jnp.full((E_PAD,), TRASH, jnp.int32).at[:E].set(edge_index[1])
    ea_pad = jnp.zeros((E_PAD, edge_attr.shape[1]), edge_attr.dtype
                       ).at[:E].set(edge_attr)

    h = _encode(x_pad, w_node, b_node, 2048)
    e = _encode(ea_pad, w_edge, b_edge, 2048)

    for (w1, b1, w2, b2) in ((c0_w1, c0_b1, c0_w2, c0_b2),
                             (c1_w1, c1_b1, c1_w2, c1_b2),
                             (c2_w1, c2_b1, c2_w2, c2_b2)):
        agg = _sc_message(h, e, src_pad, dst_pad)
        h = _conv_mlp(h, agg, w1, b1, w2, b2)

    psum, pmax, pcnt = _sc_pool(h, batch_pad)
    out = _head(psum, pmax, pcnt, h1_w, h1_b, h2_w, h2_b)
    return out.reshape(-1)
